# Initial kernel scaffold; baseline (speedup 1.0000x reference)
import jax
import jax.numpy as jnp
from jax.experimental import pallas as pl


def kernel(x, target, table):
    raise NotImplementedError("write your pallas kernel here")


# speedup vs baseline: 1.3674x; 1.3674x over previous
import functools

import jax
import jax.numpy as jnp
from jax import lax
from jax.experimental import pallas as pl
from jax.experimental.pallas import tpu as pltpu
from jax.experimental.pallas import tpu_sc as plsc

V = 1000
B = 1024
T = 50
N = B * T

NC = 2
NS = 16
NW = NC * NS
ROWS_PER_W = N // NW
CHUNK = 32
NCHUNK = ROWS_PER_W // CHUNK
L = 16


def _lse_body(table_ref, lse_ref):
    t = table_ref[...]
    m = jnp.max(t, axis=1)
    s = jnp.sum(jnp.exp(t - m[:, None]), axis=1)
    lse_ref[...] = m + jnp.log(s)


def _row_lse(table):
    return pl.pallas_call(
        _lse_body,
        out_shape=jax.ShapeDtypeStruct((V,), jnp.float32),
    )(table)


_sc_mesh = plsc.VectorSubcoreMesh(core_axis_name="c", subcore_axis_name="s")


@functools.partial(
    pl.kernel,
    mesh=_sc_mesh,
    compiler_params=pltpu.CompilerParams(
        use_tc_tiling_on_sc=False, needs_layout_passes=False),
    out_type=(
        jax.ShapeDtypeStruct((N, V), jnp.float32),
        jax.ShapeDtypeStruct((NW, L), jnp.float32),
    ),
    scratch_types=[
        pltpu.VMEM((ROWS_PER_W,), jnp.int32),
        pltpu.VMEM((ROWS_PER_W,), jnp.int32),
        pltpu.VMEM((V,), jnp.float32),
        pltpu.VMEM((CHUNK, V), jnp.float32),
        pltpu.VMEM((L,), jnp.float32),
        pltpu.SemaphoreType.DMA,
    ],
)
def _sc_gather(x_hbm, tgt_hbm, lse_hbm, table_hbm, out_hbm, loss_hbm,
               xv, tv, lsev, rows, accv, sem):
    wid = lax.axis_index("s") * NC + lax.axis_index("c")
    base = wid * ROWS_PER_W
    pltpu.sync_copy(x_hbm.at[pl.ds(base, ROWS_PER_W)], xv)
    pltpu.sync_copy(tgt_hbm.at[pl.ds(base, ROWS_PER_W)], tv)
    pltpu.sync_copy(lse_hbm, lsev)

    def chunk_body(c, acc):
        off = c * CHUNK
        pltpu.async_copy(table_hbm.at[xv.at[pl.ds(off, CHUNK)]], rows, sem).wait()
        for g in range(CHUNK // L):
            row_ids = lax.iota(jnp.int32, L) + g * L
            tg = tv[pl.ds(off + g * L, L)]
            xg = xv[pl.ds(off + g * L, L)]
            picked = plsc.load_gather(rows, [row_ids, tg])
            lx = plsc.load_gather(lsev, [xg])
            acc = acc + (lx - picked)
        pltpu.sync_copy(rows, out_hbm.at[pl.ds(base + off, CHUNK)])
        return acc

    acc = lax.fori_loop(0, NCHUNK, chunk_body, jnp.zeros((L,), jnp.float32))
    accv[...] = acc
    pltpu.sync_copy(accv, loss_hbm.at[wid])


def kernel(x, target, table):
    lse = _row_lse(table)
    logits_flat, loss_parts = _sc_gather(
        x.reshape(-1), target.reshape(-1), lse, table)
    loss = jnp.sum(loss_parts) / jnp.float32(N)
    return (logits_flat.reshape(B, T, V), loss)


# speedup vs baseline: 1.4366x; 1.0506x over previous
import functools

import jax
import jax.numpy as jnp
from jax import lax
from jax.experimental import pallas as pl
from jax.experimental.pallas import tpu as pltpu
from jax.experimental.pallas import tpu_sc as plsc

V = 1000
B = 1024
T = 50
N = B * T

NC = 2
NS = 16
NW = NC * NS
ROWS_PER_W = N // NW
CHUNK = 32
NCHUNK = ROWS_PER_W // CHUNK
L = 16


def _lse_body(table_ref, lse_ref):
    t = table_ref[...]
    m = jnp.max(t, axis=1)
    s = jnp.sum(jnp.exp(t - m[:, None]), axis=1)
    lse_ref[...] = m + jnp.log(s)


def _row_lse(table):
    return pl.pallas_call(
        _lse_body,
        out_shape=jax.ShapeDtypeStruct((V,), jnp.float32),
    )(table)


_sc_mesh = plsc.VectorSubcoreMesh(core_axis_name="c", subcore_axis_name="s")


@functools.partial(
    pl.kernel,
    mesh=_sc_mesh,
    compiler_params=pltpu.CompilerParams(
        use_tc_tiling_on_sc=False, needs_layout_passes=False),
    out_type=(
        jax.ShapeDtypeStruct((N, V), jnp.float32),
        jax.ShapeDtypeStruct((NW, L), jnp.float32),
    ),
    scratch_types=[
        pltpu.VMEM((ROWS_PER_W,), jnp.int32),
        pltpu.VMEM((ROWS_PER_W,), jnp.int32),
        pltpu.VMEM((V,), jnp.float32),
        pltpu.VMEM((CHUNK, V), jnp.float32),
        pltpu.VMEM((CHUNK, V), jnp.float32),
        pltpu.VMEM((L,), jnp.float32),
        pltpu.SemaphoreType.DMA,
        pltpu.SemaphoreType.DMA,
        pltpu.SemaphoreType.DMA,
        pltpu.SemaphoreType.DMA,
    ],
)
def _sc_gather(x_hbm, tgt_hbm, lse_hbm, table_hbm, out_hbm, loss_hbm,
               xv, tv, lsev, rows0, rows1, accv, gsem0, gsem1, ssem0, ssem1):
    bufs = (rows0, rows1)
    gsems = (gsem0, gsem1)
    ssems = (ssem0, ssem1)
    wid = lax.axis_index("s") * NC + lax.axis_index("c")
    base = wid * ROWS_PER_W
    pltpu.sync_copy(x_hbm.at[pl.ds(base, ROWS_PER_W)], xv)
    pltpu.sync_copy(tgt_hbm.at[pl.ds(base, ROWS_PER_W)], tv)
    pltpu.sync_copy(lse_hbm, lsev)

    def gather_issue(c, b):
        pltpu.async_copy(
            table_hbm.at[xv.at[pl.ds(c * CHUNK, CHUNK)]], bufs[b], gsems[b])

    def gather_wait(b):
        pltpu.make_async_copy(
            out_hbm.at[pl.ds(0, CHUNK)], bufs[b], gsems[b]).wait()

    def scatter_issue(c, b):
        pltpu.async_copy(
            bufs[b], out_hbm.at[pl.ds(base + c * CHUNK, CHUNK)], ssems[b])

    def scatter_wait(b):
        pltpu.make_async_copy(
            bufs[b], out_hbm.at[pl.ds(0, CHUNK)], ssems[b]).wait()

    gather_issue(0, 0)

    def outer_body(o, acc):
        for b in range(2):
            c = 2 * o + b
            nb = 1 - b
            if b == 0:
                @pl.when(o >= 1)
                def _():
                    scatter_wait(nb)
                gather_issue(c + 1, nb)
            else:
                scatter_wait(nb)

                @pl.when(o <= NCHUNK // 2 - 2)
                def _():
                    gather_issue(c + 1, nb)
            gather_wait(b)
            off = c * CHUNK
            for g in range(CHUNK // L):
                row_ids = lax.iota(jnp.int32, L) + g * L
                tg = tv[pl.ds(off + g * L, L)]
                xg = xv[pl.ds(off + g * L, L)]
                picked = plsc.load_gather(bufs[b], [row_ids, tg])
                lx = plsc.load_gather(lsev, [xg])
                acc = acc + (lx - picked)
            scatter_issue(c, b)
        return acc

    acc = lax.fori_loop(0, NCHUNK // 2, outer_body,
                        jnp.zeros((L,), jnp.float32))
    scatter_wait(1)
    accv[...] = acc
    pltpu.sync_copy(accv, loss_hbm.at[wid])


def kernel(x, target, table):
    lse = _row_lse(table)
    logits_flat, loss_parts = _sc_gather(
        x.reshape(-1), target.reshape(-1), lse, table)
    loss = jnp.sum(loss_parts) / jnp.float32(N)
    return (logits_flat.reshape(B, T, V), loss)
